# baseline (speedup 1.0000x reference)
import functools

import jax
import jax.numpy as jnp
from jax import lax
from jax.experimental import pallas as pl
from jax.experimental.pallas import tpu as pltpu
from jax.experimental.pallas import tpu_sc as plsc

N = 10000
E = 320000
D = 128

NC = 2
NS = 16
NW = NC * NS

CHUNK = 80
SUP = 8
NSUP = 16
CPT = SUP * NSUP
EPT = CPT * CHUNK
E_PAD = NW * EPT
ACC_N = 10240
ROWS_PER_TILE = ACC_N // NS
DUMMY_DST = N + 100
NBUF = 2

_mesh = plsc.VectorSubcoreMesh(core_axis_name="c", subcore_axis_name="s")


@functools.partial(
    pl.kernel,
    out_type=jax.ShapeDtypeStruct((NC, ACC_N, D), jnp.float32),
    mesh=_mesh,
    scratch_types=[
        pltpu.VMEM((2 * SUP, CHUNK), jnp.int32),
        pltpu.VMEM((2 * SUP, CHUNK), jnp.int32),
        pltpu.VMEM((NBUF, CHUNK, D), jnp.float32),
        pltpu.VMEM_SHARED((ACC_N, D), jnp.float32),
        pltpu.SemaphoreType.DMA((NBUF,)),
        pltpu.SemaphoreType.DMA((2,)),
        pltpu.SemaphoreType.DMA((2,)),
    ],
)
def _sc_segment_sum(x_hbm, src_hbm, dst_hbm, out_hbm,
                    src_v, dst_v, rows_v, acc_s, gsem, issem, idsem):
    cid = lax.axis_index("c")
    sid = lax.axis_index("s")
    wid = cid * NS + sid

    zv = jnp.zeros((16,), jnp.float32)

    @pl.loop(0, CHUNK)
    def _zero_fill(i):
        for k in range(D // 16):
            rows_v[0, i, pl.ds(k * 16, 16)] = zv

    for t in range(ROWS_PER_TILE // CHUNK):
        pltpu.sync_copy(rows_v.at[0],
                        acc_s.at[pl.ds(sid * ROWS_PER_TILE + t * CHUNK, CHUNK)])
    plsc.subcore_barrier()

    def _super_start(s, sb):
        pltpu.async_copy(src_hbm.at[wid, s], src_v.at[pl.ds(sb * SUP, SUP)],
                         issem.at[sb])
        pltpu.async_copy(dst_hbm.at[wid, s], dst_v.at[pl.ds(sb * SUP, SUP)],
                         idsem.at[sb])

    def _super_wait(sb):
        pltpu.make_async_copy(src_hbm.at[0, 0], src_v.at[pl.ds(sb * SUP, SUP)],
                              issem.at[sb]).wait()
        pltpu.make_async_copy(dst_hbm.at[0, 0], dst_v.at[pl.ds(sb * SUP, SUP)],
                              idsem.at[sb]).wait()

    def _gather_start(j, b, row):
        pltpu.async_copy(x_hbm.at[src_v.at[row]], rows_v.at[b], gsem.at[b])

    def _gather_wait(b):
        pltpu.make_async_copy(x_hbm.at[pl.ds(0, CHUNK)], rows_v.at[b],
                              gsem.at[b]).wait()

    _super_start(0, 0)
    _super_start(1, 1)
    _super_wait(0)
    _super_wait(1)
    _gather_start(0, 0, 0)

    @pl.loop(0, CPT, step=2 * SUP)
    def _edges(jv):
        for u in range(2 * SUP):
            j = jv + u
            b = u % NBUF
            sb = u // SUP
            row = sb * SUP + u % SUP

            if u % SUP == 2:
                @pl.when(jnp.logical_and(j >= SUP, j < CPT - SUP))
                def _():
                    _super_start(j // SUP + 1, (sb + 1) % 2)

            if u % SUP == SUP - 1:
                @pl.when(jnp.logical_and(j >= 2 * SUP - 1, j + 1 < CPT))
                def _():
                    _super_wait((sb + 1) % 2)

            _gather_wait(b)

            u2 = (u + 1) % (2 * SUP)
            row2 = (u2 // SUP) * SUP + u2 % SUP

            @pl.when(j + 1 < CPT)
            def _():
                _gather_start(j + 1, (u + 1) % NBUF, row2)

            pltpu.sync_copy(rows_v.at[b], acc_s.at[dst_v.at[row]], add=True)

    plsc.subcore_barrier()

    pltpu.sync_copy(acc_s.at[pl.ds(sid * ROWS_PER_TILE, ROWS_PER_TILE)],
                    out_hbm.at[cid, pl.ds(sid * ROWS_PER_TILE, ROWS_PER_TILE)])


_TC_BLOCK = 2000


def _mlp_body(eps_ref, x_ref, s_ref, w1_ref, b1_ref, w2_ref, b2_ref, o_ref):
    z = (1.0 + eps_ref[0]) * x_ref[...] + s_ref[0] + s_ref[1]
    h = jnp.maximum(
        jnp.dot(z, w1_ref[...], preferred_element_type=jnp.float32) + b1_ref[...],
        0.0)
    o_ref[...] = (
        jnp.dot(h, w2_ref[...], preferred_element_type=jnp.float32) + b2_ref[...])


def _tc_mlp(x, s, eps, w1, b1, w2, b2):
    return pl.pallas_call(
        _mlp_body,
        grid=(N // _TC_BLOCK,),
        in_specs=[
            pl.BlockSpec(memory_space=pltpu.SMEM),
            pl.BlockSpec((_TC_BLOCK, D), lambda i: (i, 0)),
            pl.BlockSpec((NC, _TC_BLOCK, D), lambda i: (0, i, 0)),
            pl.BlockSpec((D, D), lambda i: (0, 0)),
            pl.BlockSpec((1, D), lambda i: (0, 0)),
            pl.BlockSpec((D, D), lambda i: (0, 0)),
            pl.BlockSpec((1, D), lambda i: (0, 0)),
        ],
        out_specs=pl.BlockSpec((_TC_BLOCK, D), lambda i: (i, 0)),
        out_shape=jax.ShapeDtypeStruct((N, D), jnp.float32),
    )(eps, x, s, w1, b1, w2, b2)


def kernel(X, edge_index, eps_0, W1_0, b1_0, W2_0, b2_0,
           eps_1, W1_1, b1_1, W2_1, b2_1,
           eps_2, W1_2, b1_2, W2_2, b2_2):
    pad = E_PAD - E
    src = jnp.pad(edge_index[0], (0, pad)).reshape(NW, NSUP, SUP, CHUNK)
    dst = jnp.pad(edge_index[1], (0, pad),
                  constant_values=DUMMY_DST).reshape(NW, NSUP, SUP, CHUNK)
    params = [
        (eps_0, W1_0, b1_0, W2_0, b2_0),
        (eps_1, W1_1, b1_1, W2_1, b2_1),
        (eps_2, W1_2, b1_2, W2_2, b2_2),
    ]
    x = X
    for (eps, w1, b1, w2, b2) in params:
        s = _sc_segment_sum(x, src, dst)
        x = _tc_mlp(x, s, eps, w1, b1.reshape(1, D), w2, b2.reshape(1, D))
    return x


# speedup vs baseline: 3.6232x; 3.6232x over previous
import functools

import jax
import jax.numpy as jnp
from jax import lax
from jax.experimental import pallas as pl
from jax.experimental.pallas import tpu as pltpu
from jax.experimental.pallas import tpu_sc as plsc

N = 10000
E = 320000
D = 128

NC = 2
NS = 16
NW = NC * NS

CHUNK = 80
CPT = 125
EPT = CPT * CHUNK
ACC_N = 10240
ROWS_PER_TILE = ACC_N // NS
NBUF = 2

_mesh = plsc.VectorSubcoreMesh(core_axis_name="c", subcore_axis_name="s")


@functools.partial(
    pl.kernel,
    out_type=jax.ShapeDtypeStruct((NC, ACC_N, D), jnp.float32),
    mesh=_mesh,
    scratch_types=[
        pltpu.VMEM((EPT,), jnp.int32),
        pltpu.VMEM((CPT, CHUNK), jnp.int32),
        pltpu.VMEM((NBUF, CHUNK, D), jnp.float32),
        pltpu.VMEM_SHARED((ACC_N, D), jnp.float32),
        pltpu.SemaphoreType.DMA((NBUF,)),
    ],
)
def _sc_segment_sum(x_hbm, src_hbm, dst_hbm, out_hbm,
                    src_v, dst_v, rows_v, acc_s, gsem):
    cid = lax.axis_index("c")
    sid = lax.axis_index("s")
    wid = cid * NS + sid

    pltpu.sync_copy(src_hbm.at[wid], src_v)
    pltpu.sync_copy(dst_hbm.at[wid], dst_v)

    zv = jnp.zeros((16,), jnp.float32)

    @pl.loop(0, CHUNK)
    def _zero_fill(i):
        for k in range(D // 16):
            rows_v[0, i, pl.ds(k * 16, 16)] = zv

    for t in range(ROWS_PER_TILE // CHUNK):
        pltpu.sync_copy(rows_v.at[0],
                        acc_s.at[pl.ds(sid * ROWS_PER_TILE + t * CHUNK, CHUNK)])
    plsc.subcore_barrier()

    def _gather_start(j, b):
        pltpu.async_copy(x_hbm.at[src_v.at[pl.ds(j * CHUNK, CHUNK)]],
                         rows_v.at[b], gsem.at[b])

    def _gather_wait(j, b):
        pltpu.make_async_copy(x_hbm.at[src_v.at[pl.ds(j * CHUNK, CHUNK)]],
                              rows_v.at[b], gsem.at[b]).wait()

    _gather_start(0, 0)

    @pl.loop(0, CPT - 1, step=NBUF)
    def _edges(jv):
        for u in range(NBUF):
            j = jv + u
            b = u % NBUF
            _gather_start(j + 1, (u + 1) % NBUF)
            _gather_wait(j, b)
            pltpu.sync_copy(rows_v.at[b], acc_s.at[dst_v.at[j]], add=True)

    _gather_wait(CPT - 1, (CPT - 1) % NBUF)
    pltpu.sync_copy(rows_v.at[(CPT - 1) % NBUF], acc_s.at[dst_v.at[CPT - 1]],
                    add=True)

    plsc.subcore_barrier()

    pltpu.sync_copy(acc_s.at[pl.ds(sid * ROWS_PER_TILE, ROWS_PER_TILE)],
                    out_hbm.at[cid, pl.ds(sid * ROWS_PER_TILE, ROWS_PER_TILE)])


_TC_BLOCK = 2000


def _mlp_body(eps_ref, x_ref, s_ref, w1_ref, b1_ref, w2_ref, b2_ref, o_ref):
    z = (1.0 + eps_ref[0]) * x_ref[...] + s_ref[0] + s_ref[1]
    h = jnp.maximum(
        jnp.dot(z, w1_ref[...], preferred_element_type=jnp.float32) + b1_ref[...],
        0.0)
    o_ref[...] = (
        jnp.dot(h, w2_ref[...], preferred_element_type=jnp.float32) + b2_ref[...])


def _tc_mlp(x, s, eps, w1, b1, w2, b2):
    return pl.pallas_call(
        _mlp_body,
        grid=(N // _TC_BLOCK,),
        in_specs=[
            pl.BlockSpec(memory_space=pltpu.SMEM),
            pl.BlockSpec((_TC_BLOCK, D), lambda i: (i, 0)),
            pl.BlockSpec((NC, _TC_BLOCK, D), lambda i: (0, i, 0)),
            pl.BlockSpec((D, D), lambda i: (0, 0)),
            pl.BlockSpec((1, D), lambda i: (0, 0)),
            pl.BlockSpec((D, D), lambda i: (0, 0)),
            pl.BlockSpec((1, D), lambda i: (0, 0)),
        ],
        out_specs=pl.BlockSpec((_TC_BLOCK, D), lambda i: (i, 0)),
        out_shape=jax.ShapeDtypeStruct((N, D), jnp.float32),
    )(eps, x, s, w1, b1, w2, b2)


def kernel(X, edge_index, eps_0, W1_0, b1_0, W2_0, b2_0,
           eps_1, W1_1, b1_1, W2_1, b2_1,
           eps_2, W1_2, b1_2, W2_2, b2_2):
    src = edge_index[0].reshape(NW, EPT)
    dst = edge_index[1].reshape(NW, CPT, CHUNK)
    params = [
        (eps_0, W1_0, b1_0, W2_0, b2_0),
        (eps_1, W1_1, b1_1, W2_1, b2_1),
        (eps_2, W1_2, b1_2, W2_2, b2_2),
    ]
    x = X
    for (eps, w1, b1, w2, b2) in params:
        s = _sc_segment_sum(x, src, dst)
        x = _tc_mlp(x, s, eps, w1, b1.reshape(1, D), w2, b2.reshape(1, D))
    return x


# speedup vs baseline: 4.1444x; 1.1438x over previous
import functools

import jax
import jax.numpy as jnp
from jax import lax
from jax.experimental import pallas as pl
from jax.experimental.pallas import tpu as pltpu
from jax.experimental.pallas import tpu_sc as plsc

N = 10000
E = 320000
D = 128

NC = 2
NS = 16
NW = NC * NS

CHUNK = 80
CPT = 125
EPT = CPT * CHUNK
STRIPE = 624
NBUF = 3

_mesh = plsc.VectorSubcoreMesh(core_axis_name="c", subcore_axis_name="s")


@functools.partial(
    pl.kernel,
    out_type=jax.ShapeDtypeStruct((NC, N, D), jnp.float32),
    mesh=_mesh,
    scratch_types=[
        pltpu.VMEM((EPT,), jnp.int32),
        pltpu.VMEM((EPT,), jnp.int32),
        pltpu.VMEM((NBUF, CHUNK, D), jnp.float32),
        pltpu.VMEM_SHARED((N, D), jnp.float32),
        pltpu.SemaphoreType.DMA((NBUF,)),
        pltpu.SemaphoreType.DMA((NBUF,)),
    ],
)
def _sc_segment_sum(x_hbm, src_hbm, dst_hbm, out_hbm,
                    src_v, dst_v, rows_v, acc_s, gsem, ssem):
    cid = lax.axis_index("c")
    sid = lax.axis_index("s")
    wid = cid * NS + sid

    pltpu.sync_copy(src_hbm.at[wid], src_v)
    pltpu.sync_copy(dst_hbm.at[wid], dst_v)

    zv = jnp.zeros((16,), jnp.float32)

    @pl.loop(0, 16)
    def _zero_fill(i):
        for k in range(D // 16):
            rows_v[0, i, pl.ds(k * 16, 16)] = zv

    @pl.loop(0, STRIPE // 16)
    def _zero_acc(t):
        pltpu.sync_copy(rows_v.at[0, pl.ds(0, 16)],
                        acc_s.at[pl.ds(sid * STRIPE + t * 16, 16)])

    @pl.when(sid == NS - 1)
    def _():
        pltpu.sync_copy(rows_v.at[0, pl.ds(0, 16)],
                        acc_s.at[pl.ds(N - 16, 16)])

    plsc.subcore_barrier()

    def _gather_start(j, b):
        pltpu.async_copy(x_hbm.at[src_v.at[pl.ds(j * CHUNK, CHUNK)]],
                         rows_v.at[b], gsem.at[b])

    def _gather_wait(j, b):
        pltpu.make_async_copy(x_hbm.at[src_v.at[pl.ds(j * CHUNK, CHUNK)]],
                              rows_v.at[b], gsem.at[b]).wait()

    def _scatter_start(j, b):
        pltpu.async_copy(rows_v.at[b],
                         acc_s.at[dst_v.at[pl.ds(j * CHUNK, CHUNK)]],
                         ssem.at[b], add=True)

    def _scatter_wait(b):
        pltpu.make_async_copy(rows_v.at[b], acc_s.at[pl.ds(0, CHUNK)],
                              ssem.at[b]).wait()

    def _body(j, b):
        br = (b + 1) % NBUF

        @pl.when(j >= NBUF - 1)
        def _():
            _scatter_wait(br)

        @pl.when(j + 1 < CPT)
        def _():
            _gather_start(j + 1, br)

        _gather_wait(j, b)
        _scatter_start(j, b)

    _gather_start(0, 0)
    main = CPT - (CPT % NBUF)

    @pl.loop(0, main, step=NBUF)
    def _edges(jv):
        for u in range(NBUF):
            _body(jv + u, u)

    for j in range(main, CPT):
        _body(j, j % NBUF)
    for c in range(CPT - NBUF + 1, CPT):
        _scatter_wait(c % NBUF)

    plsc.subcore_barrier()

    pltpu.sync_copy(acc_s.at[pl.ds(sid * STRIPE, STRIPE)],
                    out_hbm.at[cid, pl.ds(sid * STRIPE, STRIPE)])

    @pl.when(sid == NS - 1)
    def _():
        pltpu.sync_copy(acc_s.at[pl.ds(NS * STRIPE, N - NS * STRIPE)],
                        out_hbm.at[cid, pl.ds(NS * STRIPE, N - NS * STRIPE)])


_TC_BLOCK = 2000


def _mlp_body(eps_ref, x_ref, s_ref, w1_ref, b1_ref, w2_ref, b2_ref, o_ref):
    z = (1.0 + eps_ref[0]) * x_ref[...] + s_ref[0] + s_ref[1]
    h = jnp.maximum(
        jnp.dot(z, w1_ref[...], preferred_element_type=jnp.float32) + b1_ref[...],
        0.0)
    o_ref[...] = (
        jnp.dot(h, w2_ref[...], preferred_element_type=jnp.float32) + b2_ref[...])


def _tc_mlp(x, s, eps, w1, b1, w2, b2):
    return pl.pallas_call(
        _mlp_body,
        grid=(N // _TC_BLOCK,),
        in_specs=[
            pl.BlockSpec(memory_space=pltpu.SMEM),
            pl.BlockSpec((_TC_BLOCK, D), lambda i: (i, 0)),
            pl.BlockSpec((NC, _TC_BLOCK, D), lambda i: (0, i, 0)),
            pl.BlockSpec((D, D), lambda i: (0, 0)),
            pl.BlockSpec((1, D), lambda i: (0, 0)),
            pl.BlockSpec((D, D), lambda i: (0, 0)),
            pl.BlockSpec((1, D), lambda i: (0, 0)),
        ],
        out_specs=pl.BlockSpec((_TC_BLOCK, D), lambda i: (i, 0)),
        out_shape=jax.ShapeDtypeStruct((N, D), jnp.float32),
    )(eps, x, s, w1, b1, w2, b2)


def kernel(X, edge_index, eps_0, W1_0, b1_0, W2_0, b2_0,
           eps_1, W1_1, b1_1, W2_1, b2_1,
           eps_2, W1_2, b1_2, W2_2, b2_2):
    src = edge_index[0].reshape(NW, EPT)
    dst = edge_index[1].reshape(NW, EPT)
    params = [
        (eps_0, W1_0, b1_0, W2_0, b2_0),
        (eps_1, W1_1, b1_1, W2_1, b2_1),
        (eps_2, W1_2, b1_2, W2_2, b2_2),
    ]
    x = X
    for (eps, w1, b1, w2, b2) in params:
        s = _sc_segment_sum(x, src, dst)
        x = _tc_mlp(x, s, eps, w1, b1.reshape(1, D), w2, b2.reshape(1, D))
    return x


# speedup vs baseline: 4.1639x; 1.0047x over previous
import functools

import jax
import jax.numpy as jnp
from jax import lax
from jax.experimental import pallas as pl
from jax.experimental.pallas import tpu as pltpu
from jax.experimental.pallas import tpu_sc as plsc

N = 10000
E = 320000
D = 128

NC = 2
NS = 16
NW = NC * NS

CHUNK = 80
CPT = 125
EPT = CPT * CHUNK
STRIPE = 624
NBUF = 3

_mesh = plsc.VectorSubcoreMesh(core_axis_name="c", subcore_axis_name="s")


@functools.partial(
    pl.kernel,
    out_type=jax.ShapeDtypeStruct((NC, N, D), jnp.float32),
    mesh=_mesh,
    scratch_types=[
        pltpu.VMEM((EPT,), jnp.int32),
        pltpu.VMEM((EPT,), jnp.int32),
        pltpu.VMEM((NBUF, CHUNK, D), jnp.float32),
        pltpu.VMEM_SHARED((N, D), jnp.float32),
        pltpu.SemaphoreType.DMA((NBUF,)),
        pltpu.SemaphoreType.DMA((NBUF,)),
    ],
)
def _sc_segment_sum(x_hbm, src_hbm, dst_hbm, out_hbm,
                    src_v, dst_v, rows_v, acc_s, gsem, ssem):
    cid = lax.axis_index("c")
    sid = lax.axis_index("s")
    wid = cid * NS + sid

    pltpu.sync_copy(src_hbm.at[wid], src_v)
    pltpu.sync_copy(dst_hbm.at[wid], dst_v)

    zv = jnp.zeros((16,), jnp.float32)

    @pl.loop(0, 16)
    def _zero_fill(i):
        for k in range(D // 16):
            rows_v[0, i, pl.ds(k * 16, 16)] = zv

    @pl.loop(0, STRIPE // 16)
    def _zero_acc(t):
        pltpu.sync_copy(rows_v.at[0, pl.ds(0, 16)],
                        acc_s.at[pl.ds(sid * STRIPE + t * 16, 16)])

    @pl.when(sid == NS - 1)
    def _():
        pltpu.sync_copy(rows_v.at[0, pl.ds(0, 16)],
                        acc_s.at[pl.ds(N - 16, 16)])

    plsc.subcore_barrier()

    def _gather_start(j, b):
        pltpu.async_copy(x_hbm.at[src_v.at[pl.ds(j * CHUNK, CHUNK)]],
                         rows_v.at[b], gsem.at[b])

    def _gather_wait(j, b):
        pltpu.make_async_copy(x_hbm.at[src_v.at[pl.ds(j * CHUNK, CHUNK)]],
                              rows_v.at[b], gsem.at[b]).wait()

    def _scatter_start(j, b):
        pltpu.async_copy(rows_v.at[b],
                         acc_s.at[dst_v.at[pl.ds(j * CHUNK, CHUNK)]],
                         ssem.at[b], add=True)

    def _scatter_wait(b):
        pltpu.make_async_copy(rows_v.at[b], acc_s.at[pl.ds(0, CHUNK)],
                              ssem.at[b]).wait()

    def _body(j, b):
        br = (b + 2) % NBUF

        _gather_wait(j, b)

        @pl.when(j >= 1)
        def _():
            _scatter_wait(br)

        @pl.when(j + 2 < CPT)
        def _():
            _gather_start(j + 2, br)

        _scatter_start(j, b)

    _gather_start(0, 0)
    _gather_start(1, 1)
    main = CPT - (CPT % NBUF)

    @pl.loop(0, main, step=NBUF)
    def _edges(jv):
        for u in range(NBUF):
            _body(jv + u, u)

    for j in range(main, CPT):
        _body(j, j % NBUF)
    _scatter_wait((CPT - 1) % NBUF)

    plsc.subcore_barrier()

    pltpu.sync_copy(acc_s.at[pl.ds(sid * STRIPE, STRIPE)],
                    out_hbm.at[cid, pl.ds(sid * STRIPE, STRIPE)])

    @pl.when(sid == NS - 1)
    def _():
        pltpu.sync_copy(acc_s.at[pl.ds(NS * STRIPE, N - NS * STRIPE)],
                        out_hbm.at[cid, pl.ds(NS * STRIPE, N - NS * STRIPE)])


_TC_BLOCK = 2000


def _mlp_body(eps_ref, x_ref, s_ref, w1_ref, b1_ref, w2_ref, b2_ref, o_ref):
    z = (1.0 + eps_ref[0]) * x_ref[...] + s_ref[0] + s_ref[1]
    h = jnp.maximum(
        jnp.dot(z, w1_ref[...], preferred_element_type=jnp.float32) + b1_ref[...],
        0.0)
    o_ref[...] = (
        jnp.dot(h, w2_ref[...], preferred_element_type=jnp.float32) + b2_ref[...])


def _tc_mlp(x, s, eps, w1, b1, w2, b2):
    return pl.pallas_call(
        _mlp_body,
        grid=(N // _TC_BLOCK,),
        in_specs=[
            pl.BlockSpec(memory_space=pltpu.SMEM),
            pl.BlockSpec((_TC_BLOCK, D), lambda i: (i, 0)),
            pl.BlockSpec((NC, _TC_BLOCK, D), lambda i: (0, i, 0)),
            pl.BlockSpec((D, D), lambda i: (0, 0)),
            pl.BlockSpec((1, D), lambda i: (0, 0)),
            pl.BlockSpec((D, D), lambda i: (0, 0)),
            pl.BlockSpec((1, D), lambda i: (0, 0)),
        ],
        out_specs=pl.BlockSpec((_TC_BLOCK, D), lambda i: (i, 0)),
        out_shape=jax.ShapeDtypeStruct((N, D), jnp.float32),
    )(eps, x, s, w1, b1, w2, b2)


def kernel(X, edge_index, eps_0, W1_0, b1_0, W2_0, b2_0,
           eps_1, W1_1, b1_1, W2_1, b2_1,
           eps_2, W1_2, b1_2, W2_2, b2_2):
    src = edge_index[0].reshape(NW, EPT)
    dst = edge_index[1].reshape(NW, EPT)
    params = [
        (eps_0, W1_0, b1_0, W2_0, b2_0),
        (eps_1, W1_1, b1_1, W2_1, b2_1),
        (eps_2, W1_2, b1_2, W2_2, b2_2),
    ]
    x = X
    for (eps, w1, b1, w2, b2) in params:
        s = _sc_segment_sum(x, src, dst)
        x = _tc_mlp(x, s, eps, w1, b1.reshape(1, D), w2, b2.reshape(1, D))
    return x


# speedup vs baseline: 4.2644x; 1.0241x over previous
import functools

import jax
import jax.numpy as jnp
from jax import lax
from jax.experimental import pallas as pl
from jax.experimental.pallas import tpu as pltpu
from jax.experimental.pallas import tpu_sc as plsc

N = 10000
E = 320000
D = 128

NC = 2
NS = 16
NW = NC * NS

CHUNK = 80
CPT = 125
EPT = CPT * CHUNK
STRIPE = 624
NBUF = 3

_mesh = plsc.VectorSubcoreMesh(core_axis_name="c", subcore_axis_name="s")


@functools.partial(
    pl.kernel,
    out_type=jax.ShapeDtypeStruct((NC, N, D), jnp.float32),
    mesh=_mesh,
    scratch_types=[
        pltpu.VMEM((EPT,), jnp.int32),
        pltpu.VMEM((EPT,), jnp.int32),
        pltpu.VMEM((NBUF, CHUNK, D), jnp.float32),
        pltpu.VMEM_SHARED((N, D), jnp.float32),
        pltpu.SemaphoreType.DMA((NBUF,)),
        pltpu.SemaphoreType.DMA((NBUF,)),
    ],
)
def _sc_segment_sum(x_hbm, src_hbm, dst_hbm, out_hbm,
                    src_v, dst_v, rows_v, acc_s, gsem, ssem):
    cid = lax.axis_index("c")
    sid = lax.axis_index("s")
    wid = cid * NS + sid

    pltpu.sync_copy(src_hbm.at[wid], src_v)
    pltpu.sync_copy(dst_hbm.at[wid], dst_v)

    def _gather_start(j, b):
        pltpu.async_copy(x_hbm.at[src_v.at[pl.ds(j * CHUNK, CHUNK)]],
                         rows_v.at[b], gsem.at[b])

    def _gather_wait(j, b):
        pltpu.make_async_copy(x_hbm.at[src_v.at[pl.ds(j * CHUNK, CHUNK)]],
                              rows_v.at[b], gsem.at[b]).wait()

    def _scatter_start(j, b):
        pltpu.async_copy(rows_v.at[b],
                         acc_s.at[dst_v.at[pl.ds(j * CHUNK, CHUNK)]],
                         ssem.at[b], add=True)

    def _scatter_wait(b):
        pltpu.make_async_copy(rows_v.at[b], acc_s.at[pl.ds(0, CHUNK)],
                              ssem.at[b]).wait()

    def _body(j, b):
        br = (b + 2) % NBUF

        _gather_wait(j, b)

        @pl.when(j >= 1)
        def _():
            _scatter_wait(br)

        @pl.when(j + 2 < CPT)
        def _():
            _gather_start(j + 2, br)

        _scatter_start(j, b)

    _gather_start(0, 0)
    _gather_start(1, 1)

    zv = jnp.zeros((16,), jnp.float32)

    @pl.loop(0, CHUNK)
    def _zero_fill(i):
        for k in range(D // 16):
            rows_v[2, i, pl.ds(k * 16, 16)] = zv

    for t in range(STRIPE // CHUNK):
        pltpu.sync_copy(rows_v.at[2],
                        acc_s.at[pl.ds(sid * STRIPE + t * CHUNK, CHUNK)])

    @pl.when(sid == NS - 1)
    def _():
        pltpu.sync_copy(rows_v.at[2], acc_s.at[pl.ds(N - CHUNK, CHUNK)])

    @pl.when(sid != NS - 1)
    def _():
        pltpu.sync_copy(
            rows_v.at[2, pl.ds(0, STRIPE - (STRIPE // CHUNK) * CHUNK)],
            acc_s.at[pl.ds(sid * STRIPE + (STRIPE // CHUNK) * CHUNK,
                           STRIPE - (STRIPE // CHUNK) * CHUNK)])

    plsc.subcore_barrier()
    main = CPT - (CPT % NBUF)

    @pl.loop(0, main, step=NBUF)
    def _edges(jv):
        for u in range(NBUF):
            _body(jv + u, u)

    for j in range(main, CPT):
        _body(j, j % NBUF)
    _scatter_wait((CPT - 1) % NBUF)

    plsc.subcore_barrier()

    pltpu.sync_copy(acc_s.at[pl.ds(sid * STRIPE, STRIPE)],
                    out_hbm.at[cid, pl.ds(sid * STRIPE, STRIPE)])

    @pl.when(sid == NS - 1)
    def _():
        pltpu.sync_copy(acc_s.at[pl.ds(NS * STRIPE, N - NS * STRIPE)],
                        out_hbm.at[cid, pl.ds(NS * STRIPE, N - NS * STRIPE)])


_TC_BLOCK = 2000


def _mlp_body(eps_ref, x_ref, s_ref, w1_ref, b1_ref, w2_ref, b2_ref, o_ref):
    z = (1.0 + eps_ref[0]) * x_ref[...] + s_ref[0] + s_ref[1]
    h = jnp.maximum(
        jnp.dot(z, w1_ref[...], preferred_element_type=jnp.float32) + b1_ref[...],
        0.0)
    o_ref[...] = (
        jnp.dot(h, w2_ref[...], preferred_element_type=jnp.float32) + b2_ref[...])


def _tc_mlp(x, s, eps, w1, b1, w2, b2):
    return pl.pallas_call(
        _mlp_body,
        grid=(N // _TC_BLOCK,),
        in_specs=[
            pl.BlockSpec(memory_space=pltpu.SMEM),
            pl.BlockSpec((_TC_BLOCK, D), lambda i: (i, 0)),
            pl.BlockSpec((NC, _TC_BLOCK, D), lambda i: (0, i, 0)),
            pl.BlockSpec((D, D), lambda i: (0, 0)),
            pl.BlockSpec((1, D), lambda i: (0, 0)),
            pl.BlockSpec((D, D), lambda i: (0, 0)),
            pl.BlockSpec((1, D), lambda i: (0, 0)),
        ],
        out_specs=pl.BlockSpec((_TC_BLOCK, D), lambda i: (i, 0)),
        out_shape=jax.ShapeDtypeStruct((N, D), jnp.float32),
    )(eps, x, s, w1, b1, w2, b2)


def kernel(X, edge_index, eps_0, W1_0, b1_0, W2_0, b2_0,
           eps_1, W1_1, b1_1, W2_1, b2_1,
           eps_2, W1_2, b1_2, W2_2, b2_2):
    src = edge_index[0].reshape(NW, EPT)
    dst = edge_index[1].reshape(NW, EPT)
    params = [
        (eps_0, W1_0, b1_0, W2_0, b2_0),
        (eps_1, W1_1, b1_1, W2_1, b2_1),
        (eps_2, W1_2, b1_2, W2_2, b2_2),
    ]
    x = X
    for (eps, w1, b1, w2, b2) in params:
        s = _sc_segment_sum(x, src, dst)
        x = _tc_mlp(x, s, eps, w1, b1.reshape(1, D), w2, b2.reshape(1, D))
    return x
